# baseline (speedup 1.0000x reference)
import functools

import jax
import jax.numpy as jnp
from jax import lax
from jax.experimental import pallas as pl
from jax.experimental.pallas import tpu as pltpu
from jax.experimental.pallas import tpu_sc as plsc

N_POINTS = 1600000
N_VOXELS = 100000
CH = 32
NCLS = 20
EPS = 1e-4

LANES = 128
NROWS = N_POINTS // LANES
NROWS_A = 12800
NROWS_C_PAD = 12512
NPAD_A = NROWS_A * LANES
VPAD = 100352
NW = 32
ROWS_PER_W = NROWS_A // NW
G = 16
GROUPS_PER_W = ROWS_PER_W // G
TILE_V = VPAD // 16

_mesh = plsc.VectorSubcoreMesh(core_axis_name="c", subcore_axis_name="s")
_sc_params = pltpu.CompilerParams(use_tc_tiling_on_sc=False)


@functools.partial(
    pl.kernel,
    out_type=jax.ShapeDtypeStruct((2 * VPAD, 8), jnp.float32),
    mesh=_mesh,
    scratch_types=[
        pltpu.VMEM((G, LANES), jnp.int32),
        pltpu.VMEM((G, LANES), jnp.int32),
        pltpu.VMEM((G * LANES, 8), jnp.float32),
        pltpu.VMEM((LANES, 8), jnp.float32),
        pltpu.VMEM_SHARED((VPAD, 8), jnp.float32),
        pltpu.SemaphoreType.DMA,
    ],
    compiler_params=_sc_params,
)
def _stage_a(xext, p2v2d, v2p2d, tmpl, zeros, out, i1, i2, vbuf, tbuf, acc,
             sem):
    c = lax.axis_index("c")
    s = lax.axis_index("s")
    wid = c * 16 + s

    pltpu.sync_copy(zeros, acc.at[pl.ds(s * TILE_V, TILE_V)])
    pltpu.sync_copy(tmpl, tbuf)
    plsc.subcore_barrier()

    def group(g, _):
        base = wid * ROWS_PER_W + g * G
        pltpu.sync_copy(p2v2d.at[pl.ds(base, G)], i1)
        pltpu.sync_copy(v2p2d.at[pl.ds(base, G)], i2)
        pltpu.sync_copy(xext.at[pl.ds(base * LANES, G * LANES)], vbuf)
        descs = []
        for j in range(G):
            descs.append(pltpu.async_copy(
                vbuf.at[pl.ds(j * LANES, LANES)], acc.at[i1.at[j]], sem,
                add=True))
            descs.append(pltpu.async_copy(tbuf, acc.at[i2.at[j]], sem,
                                          add=True))
        for d in descs:
            d.wait()
        return 0

    lax.fori_loop(0, GROUPS_PER_W, group, 0)
    plsc.subcore_barrier()

    pltpu.sync_copy(acc.at[pl.ds(s * TILE_V, TILE_V)],
                    out.at[pl.ds(c * VPAD + s * TILE_V, TILE_V)])


NCLS_PAD = 24
HALF_ROWS = VPAD // 16
REAL_ROWS = N_VOXELS // 16


def _stage_b_body(tab_ref, e6_ref, bin_ref, e7_ref, b1_ref, b2_ref, k_ref,
                  g1t, b1nt, b1vt, g2t, b2nt, b2vt, out_ref):
    f32 = jnp.float32
    P = tab_ref[:HALF_ROWS, :] + tab_ref[HALF_ROWS:, :]
    row = lax.broadcasted_iota(jnp.int32, (HALF_ROWS, 1), 0)
    rmask = row < REAL_ROWS

    def chan_sum(x):
        return jnp.dot(jnp.sum(x, axis=0, keepdims=True), k_ref[...],
                       preferred_element_type=f32)

    cntp = jnp.dot(P, e6_ref[...], preferred_element_type=f32)
    vf = P / jnp.maximum(cntp, 1.0)
    h = jnp.dot(vf, bin_ref[...], preferred_element_type=f32)
    cnt = jnp.dot(P, e7_ref[...], preferred_element_type=f32)
    cnt = jnp.where(rmask, cnt, 0.0)
    nv = jnp.float32(N_VOXELS)
    mu1 = chan_sum(h) / nv
    var1 = chan_sum(jnp.where(rmask, (h - mu1) ** 2, 0.0)) / nv
    h = jnp.maximum((h - mu1) * lax.rsqrt(var1 + EPS) * g1t[...] + b1nt[...],
                    0.0)
    z = jnp.dot(h, b1_ref[...], preferred_element_type=f32) + b1vt[...]
    n = jnp.float32(N_POINTS)
    mu2 = chan_sum(z * cnt) / n
    var2 = chan_sum(cnt * (z - mu2) ** 2) / n
    z = jnp.maximum((z - mu2) * lax.rsqrt(var2 + EPS) * g2t[...] + b2nt[...],
                    0.0)
    out_ref[...] = jnp.dot(z, b2_ref[...], preferred_element_type=f32) \
        + b2vt[...]


_stage_b = pl.pallas_call(
    _stage_b_body,
    out_shape=jax.ShapeDtypeStruct((HALF_ROWS, 16 * NCLS_PAD), jnp.float32),
)


@functools.partial(
    pl.kernel,
    out_type=jax.ShapeDtypeStruct((N_POINTS, NCLS_PAD), jnp.float32),
    mesh=_mesh,
    scratch_types=[
        pltpu.VMEM((G, LANES), jnp.int32),
        pltpu.VMEM((G * LANES, NCLS_PAD), jnp.float32),
        pltpu.SemaphoreType.DMA,
    ],
    compiler_params=_sc_params,
)
def _stage_c(table, v2p2d, out, ibuf, rows, sem):
    c = lax.axis_index("c")
    s = lax.axis_index("s")
    wid = c * 16 + s
    n_groups = jnp.where(wid < 31, GROUPS_PER_W, 6)

    def group(g, _):
        base = wid * ROWS_PER_W + g * G
        pltpu.sync_copy(v2p2d.at[pl.ds(base, G)], ibuf)
        descs = []
        for j in range(G):
            descs.append(pltpu.async_copy(
                table.at[ibuf.at[j]], rows.at[pl.ds(j * LANES, LANES)], sem))
        for d in descs:
            d.wait()
        pltpu.sync_copy(rows, out.at[pl.ds(base * LANES, G * LANES)])
        return 0

    lax.fori_loop(0, n_groups, group, 0)

    @pl.when(wid == 31)
    def _tail():
        base = NROWS - 4
        pltpu.sync_copy(v2p2d.at[pl.ds(base, G)], ibuf)
        descs = []
        for j in range(4):
            descs.append(pltpu.async_copy(
                table.at[ibuf.at[j]], rows.at[pl.ds(j * LANES, LANES)], sem))
        for d in descs:
            d.wait()
        pltpu.sync_copy(rows.at[pl.ds(0, 4 * LANES)],
                        out.at[pl.ds(base * LANES, 4 * LANES)])


_NROWS_PK = N_POINTS // 16
_NBLK = 2000


def _narrow_body(in_ref, out_ref):
    x = in_ref[...]
    out_ref[...] = jnp.concatenate(
        [x[:, NCLS_PAD * k:NCLS_PAD * k + NCLS] for k in range(16)], axis=1)


_narrow = pl.pallas_call(
    _narrow_body,
    grid=(_NROWS_PK // _NBLK,),
    in_specs=[pl.BlockSpec((_NBLK, 16 * NCLS_PAD), lambda i: (i, 0))],
    out_specs=pl.BlockSpec((_NBLK, 16 * NCLS), lambda i: (i, 0)),
    out_shape=jax.ShapeDtypeStruct((_NROWS_PK, 16 * NCLS), jnp.float32),
)


def kernel(feats, coords_float, W_in, gamma1, beta1, W1, b1, gamma2, beta2,
           W2, b2, p2v_map, v2p_map):
    f32 = jnp.float32
    p2v = p2v_map.astype(jnp.int32)
    v2p = v2p_map.astype(jnp.int32)

    ones = jnp.ones((N_POINTS, 1), f32)
    zcol = jnp.zeros((N_POINTS, 1), f32)
    xext = jnp.concatenate([feats, coords_float, ones, zcol], axis=1)
    xext = jnp.concatenate(
        [xext, jnp.zeros((NPAD_A - N_POINTS, 8), f32)], axis=0)

    pad_a = NPAD_A - N_POINTS
    p2v_a = jnp.concatenate(
        [p2v, jnp.zeros((pad_a,), jnp.int32)]).reshape(NROWS_A, LANES)
    v2p_a = jnp.concatenate(
        [v2p, jnp.full((pad_a,), N_VOXELS, jnp.int32)]).reshape(NROWS_A, LANES)

    tmpl = jnp.concatenate(
        [jnp.zeros((LANES, 7), f32), jnp.ones((LANES, 1), f32)], axis=1)
    zeros = jnp.zeros((TILE_V, 8), f32)

    tables = _stage_a(xext, p2v_a, v2p_a, tmpl, zeros)
    tab128 = tables.reshape(2 * VPAD // 16, 128)

    eye16 = jnp.eye(16, dtype=f32)
    m6 = jnp.zeros((8, 8), f32).at[6].set(1.0)
    m7 = jnp.zeros((8, CH), f32).at[7].set(1.0)
    w_in8 = jnp.concatenate([W_in, jnp.zeros((2, CH), f32)], axis=0)
    w2p = jnp.concatenate(
        [W2, jnp.zeros((CH, NCLS_PAD - NCLS), f32)], axis=1)
    b2p = jnp.concatenate([b2, jnp.zeros((NCLS_PAD - NCLS,), f32)])
    e6 = jnp.kron(eye16, m6)
    e7 = jnp.kron(eye16, m7)
    binw = jnp.kron(eye16, w_in8)
    b1w = jnp.kron(eye16, W1)
    b2w = jnp.kron(eye16, w2p)
    ksum = jnp.kron(jnp.ones((16, 16), f32), jnp.eye(CH, dtype=f32))
    tl = lambda v: jnp.tile(v, 16).reshape(1, -1)
    scores_pack = _stage_b(tab128, e6, binw, e7, b1w, b2w, ksum,
                           tl(gamma1), tl(beta1), tl(b1), tl(gamma2),
                           tl(beta2), tl(b2p))
    scores_v = scores_pack.reshape(VPAD, NCLS_PAD)

    pad_c = NROWS_C_PAD * LANES - N_POINTS
    v2p_c = jnp.concatenate(
        [v2p, jnp.zeros((pad_c,), jnp.int32)]).reshape(NROWS_C_PAD, LANES)

    out24 = _stage_c(scores_v, v2p_c)
    out = _narrow(out24.reshape(_NROWS_PK, 16 * NCLS_PAD))
    return out.reshape(N_POINTS, NCLS)


# speedup vs baseline: 1.2141x; 1.2141x over previous
import functools

import jax
import jax.numpy as jnp
from jax import lax
from jax.experimental import pallas as pl
from jax.experimental.pallas import tpu as pltpu
from jax.experimental.pallas import tpu_sc as plsc

N_POINTS = 1600000
N_VOXELS = 100000
CH = 32
NCLS = 20
EPS = 1e-4

LANES = 128
VPAD = 100352
NW = 32
PPW = N_POINTS // NW
SUBS = 15
CHUNK = SUBS * LANES
NCHUNK = PPW // CHUNK
REM = PPW - NCHUNK * CHUNK
TILE_V = VPAD // 16

_mesh = plsc.VectorSubcoreMesh(core_axis_name="c", subcore_axis_name="s")
_sc_params = pltpu.CompilerParams(use_tc_tiling_on_sc=False)


@functools.partial(
    pl.kernel,
    out_type=jax.ShapeDtypeStruct((2 * VPAD, 8), jnp.float32),
    mesh=_mesh,
    scratch_types=[
        pltpu.VMEM((SUBS, LANES), jnp.int32),
        pltpu.VMEM((SUBS, LANES), jnp.int32),
        pltpu.VMEM((CHUNK, 8), jnp.float32),
        pltpu.VMEM((LANES, 8), jnp.float32),
        pltpu.VMEM_SHARED((VPAD, 8), jnp.float32),
        pltpu.SemaphoreType.DMA,
    ],
    compiler_params=_sc_params,
)
def _stage_a(xext, p2v, v2p, tmpl, zeros, dumpc, out, i1, i2, vbuf, tbuf,
             acc, sem):
    c = lax.axis_index("c")
    s = lax.axis_index("s")
    wid = c * 16 + s

    pltpu.sync_copy(zeros, acc.at[pl.ds(s * TILE_V, TILE_V)])
    pltpu.sync_copy(tmpl, tbuf)
    plsc.subcore_barrier()

    def chunk(g, _):
        base = wid * PPW + g * CHUNK
        pltpu.sync_copy(xext.at[pl.ds(base, CHUNK)], vbuf)
        for j in range(SUBS):
            pltpu.sync_copy(p2v.at[pl.ds(base + j * LANES, LANES)], i1.at[j])
            pltpu.sync_copy(v2p.at[pl.ds(base + j * LANES, LANES)], i2.at[j])
        descs = []
        for j in range(SUBS):
            descs.append(pltpu.async_copy(
                vbuf.at[pl.ds(j * LANES, LANES)], acc.at[i1.at[j]], sem,
                add=True))
            descs.append(pltpu.async_copy(tbuf, acc.at[i2.at[j]], sem,
                                          add=True))
        for d in descs:
            d.wait()
        return 0

    lax.fori_loop(0, NCHUNK, chunk, 0)

    base = wid * PPW + NCHUNK * CHUNK
    pltpu.sync_copy(p2v.at[pl.ds(base, REM)], i1.at[0].at[pl.ds(0, REM)])
    pltpu.sync_copy(v2p.at[pl.ds(base, REM)], i2.at[0].at[pl.ds(0, REM)])
    pltpu.sync_copy(dumpc, i1.at[0].at[pl.ds(REM, LANES - REM)])
    pltpu.sync_copy(dumpc, i2.at[0].at[pl.ds(REM, LANES - REM)])
    pltpu.sync_copy(xext.at[pl.ds(base, REM)], vbuf.at[pl.ds(0, REM)])
    pltpu.sync_copy(zeros.at[pl.ds(0, LANES - REM)],
                    vbuf.at[pl.ds(REM, LANES - REM)])
    d1 = pltpu.async_copy(vbuf.at[pl.ds(0, LANES)], acc.at[i1.at[0]], sem,
                          add=True)
    d2 = pltpu.async_copy(tbuf, acc.at[i2.at[0]], sem, add=True)
    d1.wait()
    d2.wait()
    plsc.subcore_barrier()

    pltpu.sync_copy(acc.at[pl.ds(s * TILE_V, TILE_V)],
                    out.at[pl.ds(c * VPAD + s * TILE_V, TILE_V)])


NCLS_PAD = 24
HALF_ROWS = VPAD // 16
REAL_ROWS = N_VOXELS // 16


def _stage_b_body(tab_ref, e6_ref, bin_ref, e7_ref, b1_ref, b2_ref, k_ref,
                  g1t, b1nt, b1vt, g2t, b2nt, b2vt, out_ref):
    f32 = jnp.float32
    P = tab_ref[:HALF_ROWS, :] + tab_ref[HALF_ROWS:, :]
    row = lax.broadcasted_iota(jnp.int32, (HALF_ROWS, 1), 0)
    rmask = row < REAL_ROWS

    def chan_sum(x):
        return jnp.dot(jnp.sum(x, axis=0, keepdims=True), k_ref[...],
                       preferred_element_type=f32)

    cntp = jnp.dot(P, e6_ref[...], preferred_element_type=f32)
    vf = P / jnp.maximum(cntp, 1.0)
    h = jnp.dot(vf, bin_ref[...], preferred_element_type=f32)
    cnt = jnp.dot(P, e7_ref[...], preferred_element_type=f32)
    cnt = jnp.where(rmask, cnt, 0.0)
    nv = jnp.float32(N_VOXELS)
    mu1 = chan_sum(h) / nv
    var1 = chan_sum(jnp.where(rmask, (h - mu1) ** 2, 0.0)) / nv
    h = jnp.maximum((h - mu1) * lax.rsqrt(var1 + EPS) * g1t[...] + b1nt[...],
                    0.0)
    z = jnp.dot(h, b1_ref[...], preferred_element_type=f32) + b1vt[...]
    n = jnp.float32(N_POINTS)
    mu2 = chan_sum(z * cnt) / n
    var2 = chan_sum(cnt * (z - mu2) ** 2) / n
    z = jnp.maximum((z - mu2) * lax.rsqrt(var2 + EPS) * g2t[...] + b2nt[...],
                    0.0)
    out_ref[...] = jnp.dot(z, b2_ref[...], preferred_element_type=f32) \
        + b2vt[...]


_stage_b = pl.pallas_call(
    _stage_b_body,
    out_shape=jax.ShapeDtypeStruct((HALF_ROWS, 16 * NCLS_PAD), jnp.float32),
)


@functools.partial(
    pl.kernel,
    out_type=jax.ShapeDtypeStruct((N_POINTS, NCLS_PAD), jnp.float32),
    mesh=_mesh,
    scratch_types=[
        pltpu.VMEM((CHUNK,), jnp.int32),
        pltpu.VMEM((CHUNK, NCLS_PAD), jnp.float32),
        pltpu.SemaphoreType.DMA,
    ],
    compiler_params=_sc_params,
)
def _stage_c(table, v2p, out, ibuf, rows, sem):
    c = lax.axis_index("c")
    s = lax.axis_index("s")
    wid = c * 16 + s

    def chunk(g, _):
        base = wid * PPW + g * CHUNK
        pltpu.sync_copy(v2p.at[pl.ds(base, CHUNK)], ibuf)
        descs = []
        for j in range(SUBS):
            descs.append(pltpu.async_copy(
                table.at[ibuf.at[pl.ds(j * LANES, LANES)]],
                rows.at[pl.ds(j * LANES, LANES)], sem))
        for d in descs:
            d.wait()
        pltpu.sync_copy(rows, out.at[pl.ds(base, CHUNK)])
        return 0

    lax.fori_loop(0, NCHUNK, chunk, 0)

    base = wid * PPW + NCHUNK * CHUNK
    pltpu.sync_copy(v2p.at[pl.ds(base, REM)], ibuf.at[pl.ds(0, REM)])
    pltpu.async_copy(table.at[ibuf.at[pl.ds(0, REM)]],
                     rows.at[pl.ds(0, REM)], sem).wait()
    pltpu.sync_copy(rows.at[pl.ds(0, REM)], out.at[pl.ds(base, REM)])


def kernel(feats, coords_float, W_in, gamma1, beta1, W1, b1, gamma2, beta2,
           W2, b2, p2v_map, v2p_map):
    f32 = jnp.float32
    p2v = p2v_map.astype(jnp.int32)
    v2p = v2p_map.astype(jnp.int32)

    ones = jnp.ones((N_POINTS, 1), f32)
    zcol = jnp.zeros((N_POINTS, 1), f32)
    xext = jnp.concatenate([feats, coords_float, ones, zcol], axis=1)

    tmpl = jnp.concatenate(
        [jnp.zeros((LANES, 7), f32), jnp.ones((LANES, 1), f32)], axis=1)
    zeros = jnp.zeros((TILE_V, 8), f32)
    dumpc = jnp.full((LANES - REM,), N_VOXELS, jnp.int32)

    tables = _stage_a(xext, p2v, v2p, tmpl, zeros, dumpc)
    tab128 = tables.reshape(2 * VPAD // 16, 128)

    eye16 = jnp.eye(16, dtype=f32)
    m6 = jnp.zeros((8, 8), f32).at[6].set(1.0)
    m7 = jnp.zeros((8, CH), f32).at[7].set(1.0)
    w_in8 = jnp.concatenate([W_in, jnp.zeros((2, CH), f32)], axis=0)
    w2p = jnp.concatenate(
        [W2, jnp.zeros((CH, NCLS_PAD - NCLS), f32)], axis=1)
    b2p = jnp.concatenate([b2, jnp.zeros((NCLS_PAD - NCLS,), f32)])
    e6 = jnp.kron(eye16, m6)
    e7 = jnp.kron(eye16, m7)
    binw = jnp.kron(eye16, w_in8)
    b1w = jnp.kron(eye16, W1)
    b2w = jnp.kron(eye16, w2p)
    ksum = jnp.kron(jnp.ones((16, 16), f32), jnp.eye(CH, dtype=f32))
    tl = lambda v: jnp.tile(v, 16).reshape(1, -1)
    scores_pack = _stage_b(tab128, e6, binw, e7, b1w, b2w, ksum,
                           tl(gamma1), tl(beta1), tl(b1), tl(gamma2),
                           tl(beta2), tl(b2p))
    scores_v = scores_pack.reshape(VPAD, NCLS_PAD)

    out24 = _stage_c(scores_v, v2p)
    sel = jnp.zeros((NCLS_PAD, NCLS), f32).at[jnp.arange(NCLS),
                                              jnp.arange(NCLS)].set(1.0)
    return jnp.dot(out24, sel, precision=lax.Precision.HIGHEST)


# speedup vs baseline: 1.4587x; 1.2015x over previous
import functools

import jax
import jax.numpy as jnp
from jax import lax
from jax.experimental import pallas as pl
from jax.experimental.pallas import tpu as pltpu
from jax.experimental.pallas import tpu_sc as plsc

N_POINTS = 1600000
N_VOXELS = 100000
CH = 32
NCLS = 20
EPS = 1e-4

LANES = 128
VPAD = 100352
NW = 32
PPW = N_POINTS // NW
SUBS = 15
CHUNK = SUBS * LANES
NCHUNK = PPW // CHUNK
REM = PPW - NCHUNK * CHUNK
TILE_V = VPAD // 16

_mesh = plsc.VectorSubcoreMesh(core_axis_name="c", subcore_axis_name="s")
_sc_params = pltpu.CompilerParams(use_tc_tiling_on_sc=False)


@functools.partial(
    pl.kernel,
    out_type=jax.ShapeDtypeStruct((2 * VPAD, 8), jnp.float32),
    mesh=_mesh,
    scratch_types=[
        pltpu.VMEM((SUBS, LANES), jnp.int32),
        pltpu.VMEM((SUBS, LANES), jnp.int32),
        pltpu.VMEM((CHUNK, 8), jnp.float32),
        pltpu.VMEM((LANES, 8), jnp.float32),
        pltpu.VMEM_SHARED((VPAD, 8), jnp.float32),
        pltpu.SemaphoreType.DMA,
    ],
    compiler_params=_sc_params,
)
def _stage_a(xext, p2v, v2p, tmpl, zeros, dumpc, out, i1, i2, vbuf, tbuf,
             acc, sem):
    c = lax.axis_index("c")
    s = lax.axis_index("s")
    wid = c * 16 + s

    pltpu.sync_copy(zeros, acc.at[pl.ds(s * TILE_V, TILE_V)])
    pltpu.sync_copy(tmpl, tbuf)
    plsc.subcore_barrier()

    def chunk(g, _):
        base = wid * PPW + g * CHUNK
        loads = [pltpu.async_copy(xext.at[pl.ds(base, CHUNK)], vbuf, sem)]
        for j in range(SUBS):
            loads.append(pltpu.async_copy(
                p2v.at[pl.ds(base + j * LANES, LANES)], i1.at[j], sem))
            loads.append(pltpu.async_copy(
                v2p.at[pl.ds(base + j * LANES, LANES)], i2.at[j], sem))
        for d in loads:
            d.wait()
        descs = []
        for j in range(SUBS):
            descs.append(pltpu.async_copy(
                vbuf.at[pl.ds(j * LANES, LANES)], acc.at[i1.at[j]], sem,
                add=True))
            descs.append(pltpu.async_copy(tbuf, acc.at[i2.at[j]], sem,
                                          add=True))
        for d in descs:
            d.wait()
        return 0

    lax.fori_loop(0, NCHUNK, chunk, 0)

    base = wid * PPW + NCHUNK * CHUNK
    pltpu.sync_copy(p2v.at[pl.ds(base, REM)], i1.at[0].at[pl.ds(0, REM)])
    pltpu.sync_copy(v2p.at[pl.ds(base, REM)], i2.at[0].at[pl.ds(0, REM)])
    pltpu.sync_copy(dumpc, i1.at[0].at[pl.ds(REM, LANES - REM)])
    pltpu.sync_copy(dumpc, i2.at[0].at[pl.ds(REM, LANES - REM)])
    pltpu.sync_copy(xext.at[pl.ds(base, REM)], vbuf.at[pl.ds(0, REM)])
    pltpu.sync_copy(zeros.at[pl.ds(0, LANES - REM)],
                    vbuf.at[pl.ds(REM, LANES - REM)])
    d1 = pltpu.async_copy(vbuf.at[pl.ds(0, LANES)], acc.at[i1.at[0]], sem,
                          add=True)
    d2 = pltpu.async_copy(tbuf, acc.at[i2.at[0]], sem, add=True)
    d1.wait()
    d2.wait()
    plsc.subcore_barrier()

    pltpu.sync_copy(acc.at[pl.ds(s * TILE_V, TILE_V)],
                    out.at[pl.ds(c * VPAD + s * TILE_V, TILE_V)])


NCLS_PAD = 24
HALF_ROWS = VPAD // 16
REAL_ROWS = N_VOXELS // 16


def _stage_b_body(tab_ref, e6_ref, bin_ref, e7_ref, b1_ref, b2_ref, k_ref,
                  g1t, b1nt, b1vt, g2t, b2nt, b2vt, out_ref):
    f32 = jnp.float32
    P = tab_ref[:HALF_ROWS, :] + tab_ref[HALF_ROWS:, :]
    row = lax.broadcasted_iota(jnp.int32, (HALF_ROWS, 1), 0)
    rmask = row < REAL_ROWS

    def chan_sum(x):
        return jnp.dot(jnp.sum(x, axis=0, keepdims=True), k_ref[...],
                       preferred_element_type=f32)

    cntp = jnp.dot(P, e6_ref[...], preferred_element_type=f32)
    vf = P / jnp.maximum(cntp, 1.0)
    h = jnp.dot(vf, bin_ref[...], preferred_element_type=f32)
    cnt = jnp.dot(P, e7_ref[...], preferred_element_type=f32)
    cnt = jnp.where(rmask, cnt, 0.0)
    nv = jnp.float32(N_VOXELS)
    mu1 = chan_sum(h) / nv
    var1 = chan_sum(jnp.where(rmask, (h - mu1) ** 2, 0.0)) / nv
    h = jnp.maximum((h - mu1) * lax.rsqrt(var1 + EPS) * g1t[...] + b1nt[...],
                    0.0)
    z = jnp.dot(h, b1_ref[...], preferred_element_type=f32) + b1vt[...]
    n = jnp.float32(N_POINTS)
    mu2 = chan_sum(z * cnt) / n
    var2 = chan_sum(cnt * (z - mu2) ** 2) / n
    z = jnp.maximum((z - mu2) * lax.rsqrt(var2 + EPS) * g2t[...] + b2nt[...],
                    0.0)
    out_ref[...] = jnp.dot(z, b2_ref[...], preferred_element_type=f32) \
        + b2vt[...]


_stage_b = pl.pallas_call(
    _stage_b_body,
    out_shape=jax.ShapeDtypeStruct((HALF_ROWS, 16 * NCLS_PAD), jnp.float32),
)


@functools.partial(
    pl.kernel,
    out_type=jax.ShapeDtypeStruct((N_POINTS, NCLS_PAD), jnp.float32),
    mesh=_mesh,
    scratch_types=[
        pltpu.VMEM((CHUNK,), jnp.int32),
        pltpu.VMEM((CHUNK, NCLS_PAD), jnp.float32),
        pltpu.SemaphoreType.DMA,
    ],
    compiler_params=_sc_params,
)
def _stage_c(table, v2p, out, ibuf, rows, sem):
    c = lax.axis_index("c")
    s = lax.axis_index("s")
    wid = c * 16 + s

    def chunk(g, _):
        base = wid * PPW + g * CHUNK
        pltpu.sync_copy(v2p.at[pl.ds(base, CHUNK)], ibuf)
        descs = []
        for j in range(SUBS):
            descs.append(pltpu.async_copy(
                table.at[ibuf.at[pl.ds(j * LANES, LANES)]],
                rows.at[pl.ds(j * LANES, LANES)], sem))
        for d in descs:
            d.wait()
        pltpu.sync_copy(rows, out.at[pl.ds(base, CHUNK)])
        return 0

    lax.fori_loop(0, NCHUNK, chunk, 0)

    base = wid * PPW + NCHUNK * CHUNK
    pltpu.sync_copy(v2p.at[pl.ds(base, REM)], ibuf.at[pl.ds(0, REM)])
    pltpu.async_copy(table.at[ibuf.at[pl.ds(0, REM)]],
                     rows.at[pl.ds(0, REM)], sem).wait()
    pltpu.sync_copy(rows.at[pl.ds(0, REM)], out.at[pl.ds(base, REM)])


def kernel(feats, coords_float, W_in, gamma1, beta1, W1, b1, gamma2, beta2,
           W2, b2, p2v_map, v2p_map):
    f32 = jnp.float32
    p2v = p2v_map.astype(jnp.int32)
    v2p = v2p_map.astype(jnp.int32)

    ones = jnp.ones((N_POINTS, 1), f32)
    zcol = jnp.zeros((N_POINTS, 1), f32)
    xext = jnp.concatenate([feats, coords_float, ones, zcol], axis=1)

    tmpl = jnp.concatenate(
        [jnp.zeros((LANES, 7), f32), jnp.ones((LANES, 1), f32)], axis=1)
    zeros = jnp.zeros((TILE_V, 8), f32)
    dumpc = jnp.full((LANES - REM,), N_VOXELS, jnp.int32)

    tables = _stage_a(xext, p2v, v2p, tmpl, zeros, dumpc)
    tab128 = tables.reshape(2 * VPAD // 16, 128)

    eye16 = jnp.eye(16, dtype=f32)
    m6 = jnp.zeros((8, 8), f32).at[6].set(1.0)
    m7 = jnp.zeros((8, CH), f32).at[7].set(1.0)
    w_in8 = jnp.concatenate([W_in, jnp.zeros((2, CH), f32)], axis=0)
    w2p = jnp.concatenate(
        [W2, jnp.zeros((CH, NCLS_PAD - NCLS), f32)], axis=1)
    b2p = jnp.concatenate([b2, jnp.zeros((NCLS_PAD - NCLS,), f32)])
    e6 = jnp.kron(eye16, m6)
    e7 = jnp.kron(eye16, m7)
    binw = jnp.kron(eye16, w_in8)
    b1w = jnp.kron(eye16, W1)
    b2w = jnp.kron(eye16, w2p)
    ksum = jnp.kron(jnp.ones((16, 16), f32), jnp.eye(CH, dtype=f32))
    tl = lambda v: jnp.tile(v, 16).reshape(1, -1)
    scores_pack = _stage_b(tab128, e6, binw, e7, b1w, b2w, ksum,
                           tl(gamma1), tl(beta1), tl(b1), tl(gamma2),
                           tl(beta2), tl(b2p))
    scores_v = scores_pack.reshape(VPAD, NCLS_PAD)

    return _stage_c(scores_v, v2p)[:, :NCLS]
